# initial kernel scaffold (unmeasured)
import functools

import jax
import jax.numpy as jnp
from jax import lax
from jax.experimental import pallas as pl
from jax.experimental.pallas import tpu as pltpu

N_DEV = 8
SQ = 256
DM = 1024
HQ = 8
DH = 128
N_PH = 4
BLK = 64
G = 16
KV_SEL = G * BLK
SCALE = 0.08838834764831843


def _body(x_ref, wq_ref, k_hbm, v_hbm, wo_ref, out_ref,
          xg, part, rsb, ksel, vsel,
          ag_send, ag_recv, rs_send, rs_recv, cp_sem):
    my = lax.axis_index("i")
    left = lax.rem(my + N_DEV - 1, N_DEV)
    right = lax.rem(my + 1, N_DEV)
    h0 = my * HQ

    copies = []
    for p in range(N_PH):
        for g in range(G):
            kstart = (g * N_PH + p) * BLK
            for hbm, sel in ((k_hbm, ksel), (v_hbm, vsel)):
                c = pltpu.make_async_copy(
                    hbm.at[0, pl.ds(kstart, BLK), pl.ds(h0, HQ), :],
                    sel.at[p, pl.ds(g * BLK, BLK), :, :],
                    cp_sem,
                )
                c.start()
                copies.append(c)

    xg[0, :, :] = x_ref[:, :]

    barrier_sem = pltpu.get_barrier_semaphore()
    for nbr in (left, right):
        pl.semaphore_signal(barrier_sem, inc=1, device_id=(nbr,),
                            device_id_type=pl.DeviceIdType.MESH)
    pl.semaphore_wait(barrier_sem, 2)

    for h in range(N_DEV - 1):
        rdma = pltpu.make_async_remote_copy(
            src_ref=xg.at[(N_DEV - h) % N_DEV],
            dst_ref=xg.at[N_DEV - 1 - h],
            send_sem=ag_send.at[h],
            recv_sem=ag_recv.at[h],
            device_id=(right,),
            device_id_type=pl.DeviceIdType.MESH,
        )
        rdma.start()
        rdma.wait()

    for c in copies:
        c.wait()

    k = ksel[...]
    v = vsel[...]
    wq = wq_ref[...]
    wo = wo_ref[...]
    for j in range(N_DEV):
        q = jnp.dot(xg[j], wq, preferred_element_type=jnp.float32)
        q4 = q.reshape(N_PH, BLK, HQ, DH)
        s = jnp.einsum("pqhd,pkhd->phqk", q4, k,
                       preferred_element_type=jnp.float32) * SCALE
        s = s - jnp.max(s, axis=-1, keepdims=True)
        w = jnp.exp(s)
        w = w / jnp.sum(w, axis=-1, keepdims=True)
        ctx = jnp.einsum("phqk,pkhd->pqhd", w, v,
                         preferred_element_type=jnp.float32)
        part[j, :, :] = jnp.dot(ctx.reshape(SQ, HQ * DH), wo,
                                preferred_element_type=jnp.float32)

    for t in range(N_DEV - 1):
        rdma = pltpu.make_async_remote_copy(
            src_ref=part.at[N_DEV - 1 - t],
            dst_ref=rsb.at[t],
            send_sem=rs_send.at[t],
            recv_sem=rs_recv.at[t],
            device_id=(right,),
            device_id_type=pl.DeviceIdType.MESH,
        )
        rdma.start()
        rdma.wait()
        part[N_DEV - 2 - t, :, :] = part[N_DEV - 2 - t, :, :] + rsb[t, :, :]

    out_ref[:, :] = part[0, :, :]

    @functools.partial(pl.run_scoped, exit_sem=pltpu.SemaphoreType.REGULAR)
    def _(exit_sem):
        for nbr in (left, right):
            pl.semaphore_signal(exit_sem, inc=1, device_id=(nbr,),
                                device_id_type=pl.DeviceIdType.MESH)
        pl.semaphore_wait(exit_sem, 2)


def kernel(x, Wq, K_ext, V_ext, Wo):
    out = pl.pallas_call(
        _body,
        out_shape=jax.ShapeDtypeStruct((SQ, DM), jnp.float32),
        in_specs=[
            pl.BlockSpec(memory_space=pltpu.VMEM),
            pl.BlockSpec(memory_space=pltpu.VMEM),
            pl.BlockSpec(memory_space=pltpu.ANY),
            pl.BlockSpec(memory_space=pltpu.ANY),
            pl.BlockSpec(memory_space=pltpu.VMEM),
        ],
        out_specs=pl.BlockSpec(memory_space=pltpu.VMEM),
        scratch_shapes=[
            pltpu.VMEM((N_DEV, SQ, DM), jnp.float32),
            pltpu.VMEM((N_DEV, SQ, DM), jnp.float32),
            pltpu.VMEM((N_DEV - 1, SQ, DM), jnp.float32),
            pltpu.VMEM((N_PH, KV_SEL, HQ, DH), jnp.float32),
            pltpu.VMEM((N_PH, KV_SEL, HQ, DH), jnp.float32),
            pltpu.SemaphoreType.DMA((N_DEV - 1,)),
            pltpu.SemaphoreType.DMA((N_DEV - 1,)),
            pltpu.SemaphoreType.DMA((N_DEV - 1,)),
            pltpu.SemaphoreType.DMA((N_DEV - 1,)),
            pltpu.SemaphoreType.DMA,
        ],
        compiler_params=pltpu.CompilerParams(collective_id=0),
    )(x[0], Wq, K_ext, V_ext, Wo)
    return out[None]


# baseline (device time: 291159 ns/iter reference)
import functools

import jax
import jax.numpy as jnp
from jax import lax
from jax.experimental import pallas as pl
from jax.experimental.pallas import tpu as pltpu

N_DEV = 8
SQ = 256
DM = 1024
HQ = 8
DH = 128
N_PH = 4
BLK = 64
G = 16
KV_SEL = G * BLK
GRP = 2
SCALE = 0.08838834764831843


def _body(x_ref, wq_ref, k_hbm, v_hbm, wo_ref, out_ref,
          xg, rsb, kbuf, vbuf,
          ag_send, ag_recv, rs_send, rs_recv, cp_sems):
    my = lax.axis_index("i")
    left = lax.rem(my + N_DEV - 1, N_DEV)
    right = lax.rem(my + 1, N_DEV)

    def issue_phase(p):
        slot = p % 2
        cs = []
        for hbm, buf in ((k_hbm, kbuf), (v_hbm, vbuf)):
            c = pltpu.make_async_copy(hbm.at[p], buf.at[slot], cp_sems.at[slot])
            c.start()
            cs.append(c)
        return cs

    pending = {0: issue_phase(0), 1: issue_phase(1)}

    xg[0, :, :] = x_ref[:, :]

    barrier_sem = pltpu.get_barrier_semaphore()
    for nbr in (left, right):
        pl.semaphore_signal(barrier_sem, inc=1, device_id=(nbr,),
                            device_id_type=pl.DeviceIdType.MESH)
    pl.semaphore_wait(barrier_sem, 2)

    for h in range(N_DEV - 1):
        rdma = pltpu.make_async_remote_copy(
            src_ref=xg.at[(N_DEV - h) % N_DEV],
            dst_ref=xg.at[N_DEV - 1 - h],
            send_sem=ag_send.at[h],
            recv_sem=ag_recv.at[h],
            device_id=(right,),
            device_id_type=pl.DeviceIdType.MESH,
        )
        rdma.start()
        rdma.wait()

    for p in range(N_PH):
        slot = p % 2
        for c in pending.pop(p):
            c.wait()
        k = kbuf[slot]
        v = vbuf[slot]
        for g in range(N_DEV // GRP):
            rows = GRP * BLK
            xp = xg[pl.ds(GRP * g, GRP), pl.ds(p * BLK, BLK), :]
            q3 = jnp.dot(xp.reshape(rows, DM), wq_ref[...],
                         preferred_element_type=jnp.float32)
            s = jnp.einsum("qhd,hkd->hqk", q3.reshape(rows, HQ, DH), k,
                           preferred_element_type=jnp.float32) * SCALE
            s = s - jnp.max(s, axis=-1, keepdims=True)
            w = jnp.exp(s)
            w = w / jnp.sum(w, axis=-1, keepdims=True)
            ctx = jnp.einsum("hqk,hkd->qhd", w, v,
                             preferred_element_type=jnp.float32)
            pp = jnp.dot(ctx.reshape(rows, HQ * DH), wo_ref[...],
                         preferred_element_type=jnp.float32)
            xg[pl.ds(GRP * g, GRP), pl.ds(p * BLK, BLK), :] = pp.reshape(
                GRP, BLK, DM)
        if p + 2 < N_PH:
            pending[p + 2] = issue_phase(p + 2)

    for t in range(N_DEV - 1):
        rdma = pltpu.make_async_remote_copy(
            src_ref=xg.at[N_DEV - 1 - t],
            dst_ref=rsb.at[t],
            send_sem=rs_send.at[t],
            recv_sem=rs_recv.at[t],
            device_id=(right,),
            device_id_type=pl.DeviceIdType.MESH,
        )
        rdma.start()
        rdma.wait()
        xg[N_DEV - 2 - t] = xg[N_DEV - 2 - t] + rsb[t]

    out_ref[:, :] = xg[0]

    @functools.partial(pl.run_scoped, exit_sem=pltpu.SemaphoreType.REGULAR)
    def _(exit_sem):
        for nbr in (left, right):
            pl.semaphore_signal(exit_sem, inc=1, device_id=(nbr,),
                                device_id_type=pl.DeviceIdType.MESH)
        pl.semaphore_wait(exit_sem, 2)


def kernel(x, Wq, K_ext, V_ext, Wo):
    my = lax.axis_index("i")

    def sel(a):
        ah = lax.dynamic_slice_in_dim(a[0], my * HQ, HQ, axis=1)
        return ah.reshape(G, N_PH, BLK, HQ, DH).transpose(1, 3, 0, 2, 4).reshape(
            N_PH, HQ, KV_SEL, DH)

    out = pl.pallas_call(
        _body,
        out_shape=jax.ShapeDtypeStruct((SQ, DM), jnp.float32),
        in_specs=[
            pl.BlockSpec(memory_space=pltpu.VMEM),
            pl.BlockSpec(memory_space=pltpu.VMEM),
            pl.BlockSpec(memory_space=pltpu.MemorySpace.HBM),
            pl.BlockSpec(memory_space=pltpu.MemorySpace.HBM),
            pl.BlockSpec(memory_space=pltpu.VMEM),
        ],
        out_specs=pl.BlockSpec(memory_space=pltpu.VMEM),
        scratch_shapes=[
            pltpu.VMEM((N_DEV, SQ, DM), jnp.float32),
            pltpu.VMEM((N_DEV - 1, SQ, DM), jnp.float32),
            pltpu.VMEM((2, HQ, KV_SEL, DH), jnp.float32),
            pltpu.VMEM((2, HQ, KV_SEL, DH), jnp.float32),
            pltpu.SemaphoreType.DMA((N_DEV - 1,)),
            pltpu.SemaphoreType.DMA((N_DEV - 1,)),
            pltpu.SemaphoreType.DMA((N_DEV - 1,)),
            pltpu.SemaphoreType.DMA((N_DEV - 1,)),
            pltpu.SemaphoreType.DMA((2,)),
        ],
        compiler_params=pltpu.CompilerParams(
            collective_id=0, vmem_limit_bytes=100 * 1024 * 1024),
    )(x[0], Wq, sel(K_ext), sel(V_ext), Wo)
    return out[None]


# device time: 173485 ns/iter; 1.6783x vs baseline; 1.6783x over previous
import functools

import jax
import jax.numpy as jnp
from jax import lax
from jax.experimental import pallas as pl
from jax.experimental.pallas import tpu as pltpu

N_DEV = 8
SQ = 256
DM = 1024
HQ = 8
DH = 128
N_PH = 4
BLK = 64
G = 16
KV_SEL = G * BLK
SCALE = 0.08838834764831843


def _body(x_ref, wq_ref, kt_hbm, vt_hbm, wo_ref, out_ref,
          xg, part, rsb, kbf, vbf,
          ag_send, ag_recv, rs_send, rs_recv, cp_sems):
    my = lax.axis_index("i")
    left = lax.rem(my + N_DEV - 1, N_DEV)
    right = lax.rem(my + 1, N_DEV)

    kcp = pltpu.make_async_copy(kt_hbm, kbf, cp_sems.at[0])
    vcp = pltpu.make_async_copy(vt_hbm, vbf, cp_sems.at[1])
    kcp.start()
    vcp.start()

    xg[0, :, :] = x_ref[:, :]

    barrier_sem = pltpu.get_barrier_semaphore()
    for nbr in (left, right):
        pl.semaphore_signal(barrier_sem, inc=1, device_id=(nbr,),
                            device_id_type=pl.DeviceIdType.MESH)
    pl.semaphore_wait(barrier_sem, 2)

    ag = []

    def start_ag(h):
        r = pltpu.make_async_remote_copy(
            src_ref=xg.at[(N_DEV - h) % N_DEV],
            dst_ref=xg.at[N_DEV - 1 - h],
            send_sem=ag_send.at[h],
            recv_sem=ag_recv.at[h],
            device_id=(right,),
            device_id_type=pl.DeviceIdType.MESH,
        )
        r.start()
        ag.append(r)

    rs = []

    def start_rs(t):
        r = pltpu.make_async_remote_copy(
            src_ref=part.at[N_DEV - 1 - t],
            dst_ref=rsb.at[t],
            send_sem=rs_send.at[t],
            recv_sem=rs_recv.at[t],
            device_id=(right,),
            device_id_type=pl.DeviceIdType.MESH,
        )
        r.start()
        rs.append(r)

    def compute_slot(j):
        q3 = jnp.dot(xg[j], wq_ref[...],
                     preferred_element_type=jnp.float32)
        q3 = q3.astype(jnp.bfloat16).reshape(N_PH, BLK, HQ, DH)
        ctxs, sums = [], []
        for p in range(N_PH):
            s = jnp.einsum("qhd,hkd->hqk", q3[p], kbf[p],
                           preferred_element_type=jnp.float32)
            e = jnp.exp(s)
            sums.append(jnp.sum(e, axis=-1).transpose(1, 0))
            ctxs.append(jnp.einsum("hqk,hkd->qhd", e.astype(jnp.bfloat16),
                                   vbf[p],
                                   preferred_element_type=jnp.float32))
        ctx = jnp.concatenate(ctxs, axis=0)
        ssum = jnp.concatenate(sums, axis=0)
        ctx = ctx / ssum[:, :, None]
        part[j, :, :] = jnp.dot(ctx.astype(jnp.bfloat16).reshape(SQ, DM),
                                wo_ref[...],
                                preferred_element_type=jnp.float32)

    start_ag(0)
    kcp.wait()
    vcp.wait()
    compute_slot(0)

    for h in range(N_DEV - 1):
        ag[h].wait_recv()
        if h + 1 < N_DEV - 1:
            start_ag(h + 1)
        compute_slot(N_DEV - 1 - h)
        if h >= 1:
            rs[h - 1].wait_recv()
            part[N_DEV - 1 - h] = part[N_DEV - 1 - h] + rsb[h - 1]
        start_rs(h)

    rs[N_DEV - 2].wait_recv()
    out_ref[:, :] = part[0] + rsb[N_DEV - 2]

    for r in ag:
        r.wait_send()
    for r in rs:
        r.wait_send()

    @functools.partial(pl.run_scoped, exit_sem=pltpu.SemaphoreType.REGULAR)
    def _(exit_sem):
        for nbr in (left, right):
            pl.semaphore_signal(exit_sem, inc=1, device_id=(nbr,),
                                device_id_type=pl.DeviceIdType.MESH)
        pl.semaphore_wait(exit_sem, 2)


def kernel(x, Wq, K_ext, V_ext, Wo):
    my = lax.axis_index("i")

    def sel(a):
        ah = lax.dynamic_slice_in_dim(a[0], my * HQ, HQ, axis=1)
        return ah.reshape(G, N_PH, BLK, HQ, DH).transpose(1, 3, 0, 2, 4) \
                 .reshape(N_PH, HQ, KV_SEL, DH).astype(jnp.bfloat16)

    out = pl.pallas_call(
        _body,
        out_shape=jax.ShapeDtypeStruct((SQ, DM), jnp.float32),
        in_specs=[
            pl.BlockSpec(memory_space=pltpu.VMEM),
            pl.BlockSpec(memory_space=pltpu.VMEM),
            pl.BlockSpec(memory_space=pltpu.MemorySpace.HBM),
            pl.BlockSpec(memory_space=pltpu.MemorySpace.HBM),
            pl.BlockSpec(memory_space=pltpu.VMEM),
        ],
        out_specs=pl.BlockSpec(memory_space=pltpu.VMEM),
        scratch_shapes=[
            pltpu.VMEM((N_DEV, SQ, DM), jnp.bfloat16),
            pltpu.VMEM((N_DEV, SQ, DM), jnp.float32),
            pltpu.VMEM((N_DEV - 1, SQ, DM), jnp.float32),
            pltpu.VMEM((N_PH, HQ, KV_SEL, DH), jnp.bfloat16),
            pltpu.VMEM((N_PH, HQ, KV_SEL, DH), jnp.bfloat16),
            pltpu.SemaphoreType.DMA((N_DEV - 1,)),
            pltpu.SemaphoreType.DMA((N_DEV - 1,)),
            pltpu.SemaphoreType.DMA((N_DEV - 1,)),
            pltpu.SemaphoreType.DMA((N_DEV - 1,)),
            pltpu.SemaphoreType.DMA((2,)),
        ],
        compiler_params=pltpu.CompilerParams(
            collective_id=0, vmem_limit_bytes=100 * 1024 * 1024),
    )(x[0].astype(jnp.bfloat16), (Wq * SCALE).astype(jnp.bfloat16),
      sel(K_ext), sel(V_ext), Wo.astype(jnp.bfloat16))
    return out[None]


# device time: 138294 ns/iter; 2.1054x vs baseline; 1.2545x over previous
import functools

import jax
import jax.numpy as jnp
from jax import lax
from jax.experimental import pallas as pl
from jax.experimental.pallas import tpu as pltpu

N_DEV = 8
SQ = 256
DM = 1024
HQ = 8
DH = 128
N_PH = 4
BLK = 64
G = 16
KV_SEL = G * BLK
SCALE = 0.08838834764831843


def _body(x_ref, wq_ref, kt_hbm, vt_hbm, wo_ref, out_ref,
          xg, part, sbuf, rsb, kbf, vbf,
          ag_send, ag_recv, rs_send, rs_recv, cp_sems):
    my = lax.axis_index("i")
    left = lax.rem(my + N_DEV - 1, N_DEV)
    right = lax.rem(my + 1, N_DEV)

    kcp = pltpu.make_async_copy(kt_hbm, kbf, cp_sems.at[0])
    vcp = pltpu.make_async_copy(vt_hbm, vbf, cp_sems.at[1])
    kcp.start()
    vcp.start()

    xg[0, :, :] = x_ref[:, :]

    barrier_sem = pltpu.get_barrier_semaphore()
    for nbr in (left, right):
        pl.semaphore_signal(barrier_sem, inc=1, device_id=(nbr,),
                            device_id_type=pl.DeviceIdType.MESH)
    pl.semaphore_wait(barrier_sem, 2)

    ag = []

    def start_ag(h):
        r = pltpu.make_async_remote_copy(
            src_ref=xg.at[(N_DEV - h) % N_DEV],
            dst_ref=xg.at[N_DEV - 1 - h],
            send_sem=ag_send.at[h],
            recv_sem=ag_recv.at[h],
            device_id=(right,),
            device_id_type=pl.DeviceIdType.MESH,
        )
        r.start()
        ag.append(r)

    rs = []

    def start_rs(t):
        sbuf[t, :, :] = part[N_DEV - 1 - t].astype(jnp.bfloat16)
        r = pltpu.make_async_remote_copy(
            src_ref=sbuf.at[t],
            dst_ref=rsb.at[t],
            send_sem=rs_send.at[t],
            recv_sem=rs_recv.at[t],
            device_id=(right,),
            device_id_type=pl.DeviceIdType.MESH,
        )
        r.start()
        rs.append(r)

    def compute_slot(j):
        q3 = jnp.dot(xg[j], wq_ref[...],
                     preferred_element_type=jnp.float32)
        q3 = q3.astype(jnp.bfloat16).reshape(N_PH, BLK, HQ, DH)
        ctxs, sums = [], []
        for p in range(N_PH):
            s = jnp.einsum("qhd,hkd->hqk", q3[p], kbf[p],
                           preferred_element_type=jnp.float32)
            e = jnp.exp(s)
            sums.append(jnp.sum(e, axis=-1).transpose(1, 0))
            ctxs.append(jnp.einsum("hqk,hkd->qhd", e.astype(jnp.bfloat16),
                                   vbf[p],
                                   preferred_element_type=jnp.float32))
        ctx = jnp.concatenate(ctxs, axis=0)
        ssum = jnp.concatenate(sums, axis=0)
        ctx = ctx / ssum[:, :, None]
        part[j, :, :] = jnp.dot(ctx.astype(jnp.bfloat16).reshape(SQ, DM),
                                wo_ref[...],
                                preferred_element_type=jnp.float32)

    start_ag(0)
    kcp.wait()
    vcp.wait()
    compute_slot(0)

    for h in range(N_DEV - 1):
        ag[h].wait_recv()
        if h + 1 < N_DEV - 1:
            start_ag(h + 1)
        compute_slot(N_DEV - 1 - h)
        if h >= 1:
            rs[h - 1].wait_recv()
            part[N_DEV - 1 - h] = part[N_DEV - 1 - h] + rsb[h - 1]
        start_rs(h)

    rs[N_DEV - 2].wait_recv()
    out_ref[:, :] = part[0] + rsb[N_DEV - 2]

    for r in ag:
        r.wait_send()
    for r in rs:
        r.wait_send()

    @functools.partial(pl.run_scoped, exit_sem=pltpu.SemaphoreType.REGULAR)
    def _(exit_sem):
        for nbr in (left, right):
            pl.semaphore_signal(exit_sem, inc=1, device_id=(nbr,),
                                device_id_type=pl.DeviceIdType.MESH)
        pl.semaphore_wait(exit_sem, 2)


def kernel(x, Wq, K_ext, V_ext, Wo):
    my = lax.axis_index("i")

    def sel(a):
        ah = lax.dynamic_slice_in_dim(a[0], my * HQ, HQ, axis=1)
        return ah.reshape(G, N_PH, BLK, HQ, DH).transpose(1, 3, 0, 2, 4) \
                 .reshape(N_PH, HQ, KV_SEL, DH).astype(jnp.bfloat16)

    out = pl.pallas_call(
        _body,
        out_shape=jax.ShapeDtypeStruct((SQ, DM), jnp.float32),
        in_specs=[
            pl.BlockSpec(memory_space=pltpu.VMEM),
            pl.BlockSpec(memory_space=pltpu.VMEM),
            pl.BlockSpec(memory_space=pltpu.MemorySpace.HBM),
            pl.BlockSpec(memory_space=pltpu.MemorySpace.HBM),
            pl.BlockSpec(memory_space=pltpu.VMEM),
        ],
        out_specs=pl.BlockSpec(memory_space=pltpu.VMEM),
        scratch_shapes=[
            pltpu.VMEM((N_DEV, SQ, DM), jnp.bfloat16),
            pltpu.VMEM((N_DEV, SQ, DM), jnp.float32),
            pltpu.VMEM((N_DEV - 1, SQ, DM), jnp.bfloat16),
            pltpu.VMEM((N_DEV - 1, SQ, DM), jnp.bfloat16),
            pltpu.VMEM((N_PH, HQ, KV_SEL, DH), jnp.bfloat16),
            pltpu.VMEM((N_PH, HQ, KV_SEL, DH), jnp.bfloat16),
            pltpu.SemaphoreType.DMA((N_DEV - 1,)),
            pltpu.SemaphoreType.DMA((N_DEV - 1,)),
            pltpu.SemaphoreType.DMA((N_DEV - 1,)),
            pltpu.SemaphoreType.DMA((N_DEV - 1,)),
            pltpu.SemaphoreType.DMA((2,)),
        ],
        compiler_params=pltpu.CompilerParams(
            collective_id=0, vmem_limit_bytes=100 * 1024 * 1024),
    )(x[0].astype(jnp.bfloat16), (Wq * SCALE).astype(jnp.bfloat16),
      sel(K_ext), sel(V_ext), Wo.astype(jnp.bfloat16))
    return out[None]


# device time: 133857 ns/iter; 2.1751x vs baseline; 1.0331x over previous
import functools

import jax
import jax.numpy as jnp
from jax import lax
from jax.experimental import pallas as pl
from jax.experimental.pallas import tpu as pltpu

N_DEV = 8
SQ = 256
DM = 1024
HQ = 8
DH = 128
N_PH = 4
BLK = 64
G = 16
KV_SEL = G * BLK
SCALE = 0.08838834764831843


def _body(x_ref, wq_ref, kt_hbm, vt_hbm, wo_ref, out_ref,
          xg, part, sbuf, rsb, kbf, vbf,
          ag_send, ag_recv, rs_send, rs_recv, cp_sems):
    my = lax.axis_index("i")
    left = lax.rem(my + N_DEV - 1, N_DEV)
    right = lax.rem(my + 1, N_DEV)

    kcp = pltpu.make_async_copy(kt_hbm, kbf, cp_sems.at[0])
    vcp = pltpu.make_async_copy(vt_hbm, vbf, cp_sems.at[1])
    kcp.start()
    vcp.start()

    xg[0, :, :] = x_ref[:, :]

    barrier_sem = pltpu.get_barrier_semaphore()
    for nbr in (left, right):
        pl.semaphore_signal(barrier_sem, inc=1, device_id=(nbr,),
                            device_id_type=pl.DeviceIdType.MESH)
    pl.semaphore_wait(barrier_sem, 2)

    ag = []

    def start_ag(h):
        r = pltpu.make_async_remote_copy(
            src_ref=xg.at[(N_DEV - h) % N_DEV],
            dst_ref=xg.at[N_DEV - 1 - h],
            send_sem=ag_send.at[h],
            recv_sem=ag_recv.at[h],
            device_id=(right,),
            device_id_type=pl.DeviceIdType.MESH,
        )
        r.start()
        ag.append(r)

    rs = []

    def start_rs(t):
        r = pltpu.make_async_remote_copy(
            src_ref=sbuf.at[t],
            dst_ref=rsb.at[t],
            send_sem=rs_send.at[t],
            recv_sem=rs_recv.at[t],
            device_id=(right,),
            device_id_type=pl.DeviceIdType.MESH,
        )
        r.start()
        rs.append(r)

    def compute_slot(j):
        xp = xg[j]
        qh = jnp.stack(
            [jnp.dot(xp, wq_ref[:, h * DH:(h + 1) * DH],
                     preferred_element_type=jnp.float32)
             for h in range(HQ)], axis=0)
        qh = qh.astype(jnp.bfloat16)
        ctxs, sums = [], []
        for p in range(N_PH):
            s = jnp.einsum("hqd,hkd->hqk", qh[:, p * BLK:(p + 1) * BLK, :],
                           kbf[p], preferred_element_type=jnp.float32)
            e = jnp.exp(s)
            sums.append(jnp.sum(e, axis=-1))
            ctxs.append(jnp.einsum("hqk,hkd->hqd", e.astype(jnp.bfloat16),
                                   vbf[p],
                                   preferred_element_type=jnp.float32))
        ctx = jnp.concatenate(ctxs, axis=1)
        ssum = jnp.concatenate(sums, axis=1)
        ctx = (ctx / ssum[:, :, None]).astype(jnp.bfloat16)
        pp = jnp.dot(ctx[0], wo_ref[0:DH, :],
                     preferred_element_type=jnp.float32)
        for h in range(1, HQ):
            pp = pp + jnp.dot(ctx[h], wo_ref[h * DH:(h + 1) * DH, :],
                              preferred_element_type=jnp.float32)
        part[j, :, :] = pp

    start_ag(0)
    kcp.wait()
    vcp.wait()
    compute_slot(0)

    for h in range(N_DEV - 1):
        ag[h].wait_recv()
        if h + 1 < N_DEV - 1:
            start_ag(h + 1)
        compute_slot(N_DEV - 1 - h)
        if h == 0:
            sbuf[0, :, :] = part[N_DEV - 1].astype(jnp.bfloat16)
        else:
            rs[h - 1].wait_recv()
            sbuf[h, :, :] = (part[N_DEV - 1 - h]
                             + rsb[h - 1]).astype(jnp.bfloat16)
        start_rs(h)

    rs[N_DEV - 2].wait_recv()
    out_ref[:, :] = part[0] + rsb[N_DEV - 2]

    for r in ag:
        r.wait_send()
    for r in rs:
        r.wait_send()

    @functools.partial(pl.run_scoped, exit_sem=pltpu.SemaphoreType.REGULAR)
    def _(exit_sem):
        for nbr in (left, right):
            pl.semaphore_signal(exit_sem, inc=1, device_id=(nbr,),
                                device_id_type=pl.DeviceIdType.MESH)
        pl.semaphore_wait(exit_sem, 2)


def kernel(x, Wq, K_ext, V_ext, Wo):
    my = lax.axis_index("i")

    def sel(a):
        ah = lax.dynamic_slice_in_dim(a[0], my * HQ, HQ, axis=1)
        return ah.reshape(G, N_PH, BLK, HQ, DH).transpose(1, 3, 0, 2, 4) \
                 .reshape(N_PH, HQ, KV_SEL, DH).astype(jnp.bfloat16)

    out = pl.pallas_call(
        _body,
        out_shape=jax.ShapeDtypeStruct((SQ, DM), jnp.float32),
        in_specs=[
            pl.BlockSpec(memory_space=pltpu.VMEM),
            pl.BlockSpec(memory_space=pltpu.VMEM),
            pl.BlockSpec(memory_space=pltpu.MemorySpace.HBM),
            pl.BlockSpec(memory_space=pltpu.MemorySpace.HBM),
            pl.BlockSpec(memory_space=pltpu.VMEM),
        ],
        out_specs=pl.BlockSpec(memory_space=pltpu.VMEM),
        scratch_shapes=[
            pltpu.VMEM((N_DEV, SQ, DM), jnp.bfloat16),
            pltpu.VMEM((N_DEV, SQ, DM), jnp.float32),
            pltpu.VMEM((N_DEV - 1, SQ, DM), jnp.bfloat16),
            pltpu.VMEM((N_DEV - 1, SQ, DM), jnp.bfloat16),
            pltpu.VMEM((N_PH, HQ, KV_SEL, DH), jnp.bfloat16),
            pltpu.VMEM((N_PH, HQ, KV_SEL, DH), jnp.bfloat16),
            pltpu.SemaphoreType.DMA((N_DEV - 1,)),
            pltpu.SemaphoreType.DMA((N_DEV - 1,)),
            pltpu.SemaphoreType.DMA((N_DEV - 1,)),
            pltpu.SemaphoreType.DMA((N_DEV - 1,)),
            pltpu.SemaphoreType.DMA((2,)),
        ],
        compiler_params=pltpu.CompilerParams(
            collective_id=0, vmem_limit_bytes=100 * 1024 * 1024),
    )(x[0].astype(jnp.bfloat16), (Wq * SCALE).astype(jnp.bfloat16),
      sel(K_ext), sel(V_ext), Wo.astype(jnp.bfloat16))
    return out[None]


# device time: 118343 ns/iter; 2.4603x vs baseline; 1.1311x over previous
import functools

import jax
import jax.numpy as jnp
from jax import lax
from jax.experimental import pallas as pl
from jax.experimental.pallas import tpu as pltpu

N_DEV = 8
SQ = 256
DM = 1024
HQ = 8
DH = 128
N_PH = 4
BLK = 64
G = 16
KV_SEL = G * BLK
NR = 4
NL = 3
SCALE = 0.08838834764831843
BF = jnp.bfloat16


def _body(x_ref, wq_ref, kt_hbm, vt_hbm, wo_ref, out_ref,
          xg, part, sbr, rbr, sbl, rbl, kbf, vbf,
          agr_s, agr_r, agl_s, agl_r,
          rsr_s, rsr_r, rsl_s, rsl_r, cp_sems):
    my = lax.axis_index("i")
    left = lax.rem(my + N_DEV - 1, N_DEV)
    right = lax.rem(my + 1, N_DEV)

    kcp = pltpu.make_async_copy(kt_hbm, kbf, cp_sems.at[0])
    vcp = pltpu.make_async_copy(vt_hbm, vbf, cp_sems.at[1])
    kcp.start()
    vcp.start()

    xg[0, :, :] = x_ref[:, :]

    barrier_sem = pltpu.get_barrier_semaphore()
    for nbr in (left, right):
        pl.semaphore_signal(barrier_sem, inc=1, device_id=(nbr,),
                            device_id_type=pl.DeviceIdType.MESH)
    pl.semaphore_wait(barrier_sem, 2)

    sends = []

    def rdma(src, dst, ssem, rsem, dev):
        r = pltpu.make_async_remote_copy(
            src_ref=src, dst_ref=dst, send_sem=ssem, recv_sem=rsem,
            device_id=(dev,), device_id_type=pl.DeviceIdType.MESH)
        r.start()
        sends.append(r)
        return r

    agr = [None] * NR
    def start_agr(h):
        agr[h] = rdma(xg.at[(N_DEV - h) % N_DEV], xg.at[N_DEV - 1 - h],
                      agr_s.at[h], agr_r.at[h], right)

    agl = [None] * NL
    def start_agl(h):
        agl[h] = rdma(xg.at[h], xg.at[h + 1], agl_s.at[h], agl_r.at[h], left)

    def compute_slot(j):
        xp = xg[j]
        qh = jnp.stack(
            [jnp.dot(xp, wq_ref[:, h * DH:(h + 1) * DH],
                     preferred_element_type=jnp.float32)
             for h in range(HQ)], axis=0).astype(BF)
        ctxs, sums = [], []
        for p in range(N_PH):
            s = jnp.einsum("hqd,hkd->hqk", qh[:, p * BLK:(p + 1) * BLK, :],
                           kbf[p], preferred_element_type=jnp.float32)
            e = jnp.exp(s)
            sums.append(jnp.sum(e, axis=-1))
            ctxs.append(jnp.einsum("hqk,hkd->hqd", e.astype(BF), vbf[p],
                                   preferred_element_type=jnp.float32))
        ctx = jnp.concatenate(ctxs, axis=1)
        ssum = jnp.concatenate(sums, axis=1)
        ctx = (ctx / ssum[:, :, None]).astype(BF)
        pp = jnp.dot(ctx[0], wo_ref[0:DH, :],
                     preferred_element_type=jnp.float32)
        for h in range(1, HQ):
            pp = pp + jnp.dot(ctx[h], wo_ref[h * DH:(h + 1) * DH, :],
                              preferred_element_type=jnp.float32)
        part[j, :, :] = pp

    start_agr(0)
    start_agl(0)
    kcp.wait()
    vcp.wait()
    compute_slot(0)

    agr[0].wait_recv()
    start_agr(1)
    agl[0].wait_recv()
    start_agl(1)
    compute_slot(7)

    agr[1].wait_recv()
    start_agr(2)
    agl[1].wait_recv()
    start_agl(2)
    compute_slot(6)

    agr[2].wait_recv()
    start_agr(3)
    agl[2].wait_recv()
    compute_slot(5)

    rsl = [None] * NL
    sbl[0, :, :] = part[5].astype(BF)
    rsl[0] = rdma(sbl.at[0], rbl.at[0], rsl_s.at[0], rsl_r.at[0], left)

    agr[3].wait_recv()
    compute_slot(4)

    rsr = [None] * NR
    sbr[0, :, :] = part[4].astype(BF)
    rsr[0] = rdma(sbr.at[0], rbr.at[0], rsr_s.at[0], rsr_r.at[0], right)

    compute_slot(3)
    rsl[0].wait_recv()
    sbl[1, :, :] = (part[6] + rbl[0]).astype(BF)
    rsl[1] = rdma(sbl.at[1], rbl.at[1], rsl_s.at[1], rsl_r.at[1], left)
    rsr[0].wait_recv()
    sbr[1, :, :] = (part[3] + rbr[0]).astype(BF)
    rsr[1] = rdma(sbr.at[1], rbr.at[1], rsr_s.at[1], rsr_r.at[1], right)

    compute_slot(2)
    rsl[1].wait_recv()
    sbl[2, :, :] = (part[7] + rbl[1]).astype(BF)
    rsl[2] = rdma(sbl.at[2], rbl.at[2], rsl_s.at[2], rsl_r.at[2], left)
    rsr[1].wait_recv()
    sbr[2, :, :] = (part[2] + rbr[1]).astype(BF)
    rsr[2] = rdma(sbr.at[2], rbr.at[2], rsr_s.at[2], rsr_r.at[2], right)

    compute_slot(1)
    rsr[2].wait_recv()
    sbr[3, :, :] = (part[1] + rbr[2]).astype(BF)
    rsr[3] = rdma(sbr.at[3], rbr.at[3], rsr_s.at[3], rsr_r.at[3], right)

    rsl[2].wait_recv()
    rsr[3].wait_recv()
    out_ref[:, :] = part[0] + rbl[2] + rbr[3]

    for r in sends:
        r.wait_send()

    @functools.partial(pl.run_scoped, exit_sem=pltpu.SemaphoreType.REGULAR)
    def _(exit_sem):
        for nbr in (left, right):
            pl.semaphore_signal(exit_sem, inc=1, device_id=(nbr,),
                                device_id_type=pl.DeviceIdType.MESH)
        pl.semaphore_wait(exit_sem, 2)


def kernel(x, Wq, K_ext, V_ext, Wo):
    my = lax.axis_index("i")

    def sel(a):
        ah = lax.dynamic_slice_in_dim(a[0], my * HQ, HQ, axis=1)
        return ah.reshape(G, N_PH, BLK, HQ, DH).transpose(1, 3, 0, 2, 4) \
                 .reshape(N_PH, HQ, KV_SEL, DH).astype(BF)

    out = pl.pallas_call(
        _body,
        out_shape=jax.ShapeDtypeStruct((SQ, DM), jnp.float32),
        in_specs=[
            pl.BlockSpec(memory_space=pltpu.VMEM),
            pl.BlockSpec(memory_space=pltpu.VMEM),
            pl.BlockSpec(memory_space=pltpu.MemorySpace.HBM),
            pl.BlockSpec(memory_space=pltpu.MemorySpace.HBM),
            pl.BlockSpec(memory_space=pltpu.VMEM),
        ],
        out_specs=pl.BlockSpec(memory_space=pltpu.VMEM),
        scratch_shapes=[
            pltpu.VMEM((N_DEV, SQ, DM), BF),
            pltpu.VMEM((N_DEV, SQ, DM), jnp.float32),
            pltpu.VMEM((NR, SQ, DM), BF),
            pltpu.VMEM((NR, SQ, DM), BF),
            pltpu.VMEM((NL, SQ, DM), BF),
            pltpu.VMEM((NL, SQ, DM), BF),
            pltpu.VMEM((N_PH, HQ, KV_SEL, DH), BF),
            pltpu.VMEM((N_PH, HQ, KV_SEL, DH), BF),
            pltpu.SemaphoreType.DMA((NR,)),
            pltpu.SemaphoreType.DMA((NR,)),
            pltpu.SemaphoreType.DMA((NL,)),
            pltpu.SemaphoreType.DMA((NL,)),
            pltpu.SemaphoreType.DMA((NR,)),
            pltpu.SemaphoreType.DMA((NR,)),
            pltpu.SemaphoreType.DMA((NL,)),
            pltpu.SemaphoreType.DMA((NL,)),
            pltpu.SemaphoreType.DMA((2,)),
        ],
        compiler_params=pltpu.CompilerParams(
            collective_id=0, vmem_limit_bytes=100 * 1024 * 1024),
    )(x[0].astype(BF), (Wq * SCALE).astype(BF),
      sel(K_ext), sel(V_ext), Wo.astype(BF))
    return out[None]
